# baseline (speedup 1.0000x reference)
import jax
import jax.numpy as jnp
from jax import lax
from jax.experimental import pallas as pl
from jax.experimental.pallas import tpu as pltpu

B, S, D = 4, 8192, 2048
H = D // 2
E = 64
BULK = 2048
TAIL = 512
T2 = 4
T1 = (B * S - T2 * TAIL) // BULK
TAIL_OFF = T1 * BULK // TAIL
BPB = S // BULK


def _router_kernel(xb_ref, xt_ref, w1_ref, b1_ref, w2_ref, b2_ref,
                   out_ref, acc_ref):
    t = pl.program_id(0)

    @pl.when(t < T1)
    def _bulk():
        part = jnp.sum(xb_ref[...], axis=0, keepdims=True)
        b = t // BPB

        @pl.when(t % BPB == 0)
        def _init():
            acc_ref[pl.ds(b, 1), :] = part

        @pl.when(t % BPB != 0)
        def _accum():
            acc_ref[pl.ds(b, 1), :] += part

    @pl.when(t >= T1)
    def _tail():
        part = jnp.sum(xt_ref[...], axis=0, keepdims=True)
        acc_ref[pl.ds(B - 1, 1), :] += part

    @pl.when(t == T1 + T2 - 1)
    def _finish():
        pooled = acc_ref[...] * (1.0 / S)
        h = lax.dot_general(
            pooled, w1_ref[...], (((1,), (0,)), ((), ())),
            preferred_element_type=jnp.float32,
        ) + b1_ref[...]
        h = 0.5 * h * (1.0 + lax.erf(h * (2.0 ** -0.5)))
        logits = lax.dot_general(
            h, w2_ref[...], (((1,), (0,)), ((), ())),
            preferred_element_type=jnp.float32,
        ) + b2_ref[...]
        m = jnp.max(logits, axis=-1, keepdims=True)
        e = jnp.exp(logits - m)
        out_ref[...] = e / jnp.sum(e, axis=-1, keepdims=True)


@jax.jit
def kernel(x, W1, b1, W2, b2):
    xf = x.reshape(B * S, D)
    out = pl.pallas_call(
        _router_kernel,
        grid=(T1 + T2,),
        in_specs=[
            pl.BlockSpec((BULK, D), lambda t: (jnp.minimum(t, T1 - 1), 0)),
            pl.BlockSpec((TAIL, D),
                         lambda t: (jnp.maximum(t - T1, 0) + TAIL_OFF, 0)),
            pl.BlockSpec((D, H), lambda t: (0, 0)),
            pl.BlockSpec((1, H), lambda t: (0, 0)),
            pl.BlockSpec((H, E), lambda t: (0, 0)),
            pl.BlockSpec((1, E), lambda t: (0, 0)),
        ],
        out_specs=pl.BlockSpec((B, E), lambda t: (0, 0)),
        out_shape=jax.ShapeDtypeStruct((B, E), jnp.float32),
        scratch_shapes=[pltpu.VMEM((B, D), jnp.float32)],
        compiler_params=pltpu.CompilerParams(
            dimension_semantics=("arbitrary",),
        ),
    )(xf, xf, W1, b1.reshape(1, H), W2, b2.reshape(1, E))
    return out


# speedup vs baseline: 1.0269x; 1.0269x over previous
import jax
import jax.numpy as jnp
from jax import lax
from jax.experimental import pallas as pl
from jax.experimental.pallas import tpu as pltpu

B, S, D = 4, 8192, 2048
H = D // 2
E = 64
S_BLK = 1024


def _router_kernel(x_ref, w1_ref, b1_ref, w2_ref, b2_ref, out_ref, acc_ref):
    b = pl.program_id(0)
    j = pl.program_id(1)
    nj = pl.num_programs(1)

    part = jnp.sum(x_ref[0], axis=0, keepdims=True)

    @pl.when(j == 0)
    def _init():
        acc_ref[pl.ds(b, 1), :] = part

    @pl.when(j > 0)
    def _accum():
        acc_ref[pl.ds(b, 1), :] += part

    @pl.when((b == B - 1) & (j == nj - 1))
    def _finish():
        pooled = acc_ref[...] * (1.0 / S)
        h = lax.dot_general(
            pooled, w1_ref[...], (((1,), (0,)), ((), ())),
            preferred_element_type=jnp.float32,
        ) + b1_ref[...]
        h = 0.5 * h * (1.0 + lax.erf(h * (2.0 ** -0.5)))
        logits = lax.dot_general(
            h, w2_ref[...], (((1,), (0,)), ((), ())),
            preferred_element_type=jnp.float32,
        ) + b2_ref[...]
        m = jnp.max(logits, axis=-1, keepdims=True)
        e = jnp.exp(logits - m)
        out_ref[...] = e / jnp.sum(e, axis=-1, keepdims=True)


@jax.jit
def kernel(x, W1, b1, W2, b2):
    grid = (B, S // S_BLK)
    out = pl.pallas_call(
        _router_kernel,
        grid=grid,
        in_specs=[
            pl.BlockSpec((1, S_BLK, D), lambda b, j: (b, j, 0)),
            pl.BlockSpec((D, H), lambda b, j: (0, 0)),
            pl.BlockSpec((1, H), lambda b, j: (0, 0)),
            pl.BlockSpec((H, E), lambda b, j: (0, 0)),
            pl.BlockSpec((1, E), lambda b, j: (0, 0)),
        ],
        out_specs=pl.BlockSpec((B, E), lambda b, j: (0, 0)),
        out_shape=jax.ShapeDtypeStruct((B, E), jnp.float32),
        scratch_shapes=[pltpu.VMEM((B, D), jnp.float32)],
        compiler_params=pltpu.CompilerParams(
            dimension_semantics=("arbitrary", "arbitrary"),
        ),
    )(x, W1, b1.reshape(1, H), W2, b2.reshape(1, E))
    return out
